# Initial kernel scaffold; baseline (speedup 1.0000x reference)
import jax
import jax.numpy as jnp
from jax.experimental import pallas as pl


def kernel(x, edge_attr, edge_index, batch, params):
    raise NotImplementedError("write your pallas kernel here")


# speedup vs baseline: 1.2512x; 1.2512x over previous
import functools

import jax
import jax.numpy as jnp
from jax import lax
from jax.experimental import pallas as pl
from jax.experimental.pallas import tpu as pltpu
from jax.experimental.pallas import tpu_sc as plsc

N = 50000
E = 800000
FN = 9
FE = 3
H = 64
B = 256
TASKS = 12
BN_EPS = 1e-5

NC, NS = 2, 16
N_PAD = 50176
E_PAD = 819200
CHUNK = 256
GRP = CHUNK // 128
ROWS_T = N_PAD // (NC * NS)
PCH = 224
SEG = 264
NEG = -3.4028235e38

_f32 = jnp.float32


def _sc_mesh():
  return plsc.VectorSubcoreMesh(
      core_axis_name="c", subcore_axis_name="s", num_cores=NC, num_subcores=NS
  )


D = 32
PER_SUB = E_PAD // NS
N_CHUNKS = PER_SUB // CHUNK
ROWS_SUB = N_PAD // NS


def _edge_body(tab, eproj, src1, dst1, out, sidx0, sidx1, didx0, didx1,
               xg, ev, acc, gsem, ssem):
  c = lax.axis_index("c")
  s = lax.axis_index("s")

  zero = jnp.zeros((16,), _f32)

  @pl.loop(0, CHUNK)
  def _(i):
    for j in range(D // 16):
      xg[i, pl.ds(j * 16, 16)] = zero

  row0 = pl.multiple_of(s * ROWS_SUB, 64)
  nf = ROWS_SUB // CHUNK
  rem = ROWS_SUB - nf * CHUNK
  for k in range(nf):
    pltpu.sync_copy(xg, acc.at[pl.ds(row0 + k * CHUNK, CHUNK)])
  if rem:
    pltpu.sync_copy(
        xg.at[pl.ds(0, rem)], acc.at[pl.ds(row0 + nf * CHUNK, rem)]
    )
  plsc.subcore_barrier()

  tabc = tab.at[c]
  ec_ref = eproj.at[c]
  ebase = pl.multiple_of(s * PER_SUB, 1024)

  @pl.loop(0, N_CHUNKS)
  def _(k):
    eoff = pl.multiple_of(ebase + k * CHUNK, CHUNK)
    pltpu.sync_copy(src1.at[pl.ds(eoff, 128)], sidx0)
    pltpu.sync_copy(src1.at[pl.ds(eoff + 128, 128)], sidx1)
    pltpu.sync_copy(dst1.at[pl.ds(eoff, 128)], didx0)
    pltpu.sync_copy(dst1.at[pl.ds(eoff + 128, 128)], didx1)
    pltpu.sync_copy(ec_ref.at[pl.ds(eoff, CHUNK)], ev)
    for j, sidx in ((0, sidx0), (1, sidx1)):
      pltpu.make_async_copy(
          tabc.at[sidx], xg.at[pl.ds(j * 128, 128)], gsem
      ).start()
      pltpu.make_async_copy(
          tabc.at[sidx], xg.at[pl.ds(j * 128, 128)], gsem
      ).wait()

    @pl.loop(0, CHUNK, unroll=8)
    def _(i):
      for j in range(D // 16):
        sl = pl.ds(j * 16, 16)
        ev[i, sl] = jnp.maximum(xg[i, sl] + ev[i, sl], 0.0)

    for j, didx in ((0, didx0), (1, didx1)):
      pltpu.make_async_copy(
          ev.at[pl.ds(j * 128, 128)], acc.at[didx], ssem
      ).start(add=True)
      pltpu.make_async_copy(
          ev.at[pl.ds(j * 128, 128)], acc.at[didx], ssem
      ).wait()

  plsc.subcore_barrier()
  pltpu.sync_copy(
      acc.at[pl.ds(row0, ROWS_SUB)],
      out.at[c].at[pl.ds(row0, ROWS_SUB)],
  )


_edge_sc = pl.kernel(
    _edge_body,
    out_type=jax.ShapeDtypeStruct((NC, N_PAD, D), _f32),
    mesh=_sc_mesh(),
    compiler_params=pltpu.CompilerParams(use_tc_tiling_on_sc=False),
    scratch_types=[
        pltpu.VMEM((128,), jnp.int32),
        pltpu.VMEM((128,), jnp.int32),
        pltpu.VMEM((128,), jnp.int32),
        pltpu.VMEM((128,), jnp.int32),
        pltpu.VMEM((CHUNK, D), _f32),
        pltpu.VMEM((CHUNK, D), _f32),
        pltpu.VMEM_SHARED((N_PAD, D), _f32),
        pltpu.SemaphoreType.DMA,
        pltpu.SemaphoreType.DMA,
    ],
)


def _pool1(h, gate, batch2):
  def body(hh, gg, bb, osum, omax, ogmax, ocnt, hv, gv, bv, sacc, macc, gacc,
           cacc):
    c = lax.axis_index("c")
    s = lax.axis_index("s")
    wid = c * NS + s
    base = wid * ROWS_T

    zero = jnp.zeros((16,), _f32)
    neg = jnp.full((16,), NEG, _f32)

    @pl.loop(0, SEG)
    def _(i):
      for j in range(4):
        sacc[i, pl.ds(j * 16, 16)] = zero
        macc[i, pl.ds(j * 16, 16)] = neg
      cacc[i, pl.ds(0, 16)] = zero
      gacc[i, pl.ds(0, 16)] = neg

    @pl.loop(0, ROWS_T // PCH)
    def _(k):
      off = pl.multiple_of(base + k * PCH, 32)
      pltpu.sync_copy(hh.at[pl.ds(off, PCH)], hv)
      pltpu.sync_copy(gg.at[pl.ds(off, PCH)], gv)
      pltpu.sync_copy(bb.at[pl.ds(off, PCH)], bv)

      @pl.loop(0, PCH // 16)
      def _(q):
        bvec = bv[pl.ds(q * 16, 16)]
        gvec = gv[pl.ds(q * 16, 16)]
        for t in range(16):
          i = q * 16 + t
          b = bvec[t]
          for j in range(4):
            sl = pl.ds(j * 16, 16)
            hvec = hv[i, sl]
            sacc[b, sl] = sacc[b, sl] + hvec
            macc[b, sl] = jnp.maximum(macc[b, sl], hvec)
          c16 = pl.ds(0, 16)
          gvb = jnp.full((16,), gvec[t], _f32)
          gacc[b, c16] = jnp.maximum(gacc[b, c16], gvb)
          cacc[b, c16] = cacc[b, c16] + 1.0

    woff = pl.multiple_of(wid * SEG, 8)
    pltpu.sync_copy(sacc, osum.at[pl.ds(woff, SEG)])
    pltpu.sync_copy(macc, omax.at[pl.ds(woff, SEG)])
    pltpu.sync_copy(gacc, ogmax.at[pl.ds(woff, SEG)])
    pltpu.sync_copy(cacc, ocnt.at[pl.ds(woff, SEG)])

  f = pl.kernel(
      body,
      out_type=(
          jax.ShapeDtypeStruct((NC * NS * SEG, H), _f32),
          jax.ShapeDtypeStruct((NC * NS * SEG, H), _f32),
          jax.ShapeDtypeStruct((NC * NS * SEG, 16), _f32),
          jax.ShapeDtypeStruct((NC * NS * SEG, 16), _f32),
      ),
      mesh=_sc_mesh(),
      compiler_params=pltpu.CompilerParams(use_tc_tiling_on_sc=False),
      scratch_types=[
          pltpu.VMEM((PCH, H), _f32),
          pltpu.VMEM((PCH,), _f32),
          pltpu.VMEM((PCH,), jnp.int32),
          pltpu.VMEM((SEG, H), _f32),
          pltpu.VMEM((SEG, H), _f32),
          pltpu.VMEM((SEG, 16), _f32),
          pltpu.VMEM((SEG, 16), _f32),
      ],
  )
  return f(h, gate, batch2)


def _pool2(h, gate, batch2, gmax):
  def body(hh, gg, bb, gm, oattn, oden, hv, gv, bv, gmv, aacc, dacc):
    c = lax.axis_index("c")
    s = lax.axis_index("s")
    wid = c * NS + s
    base = wid * ROWS_T

    zero = jnp.zeros((16,), _f32)

    @pl.loop(0, SEG)
    def _(i):
      for j in range(4):
        aacc[i, pl.ds(j * 16, 16)] = zero
      dacc[i, pl.ds(0, 16)] = zero

    pltpu.sync_copy(gm, gmv)

    @pl.loop(0, ROWS_T // PCH)
    def _(k):
      off = pl.multiple_of(base + k * PCH, 32)
      pltpu.sync_copy(hh.at[pl.ds(off, PCH)], hv)
      pltpu.sync_copy(gg.at[pl.ds(off, PCH)], gv)
      pltpu.sync_copy(bb.at[pl.ds(off, PCH)], bv)

      @pl.loop(0, PCH // 16)
      def _(q):
        bvec = bv[pl.ds(q * 16, 16)]
        gvec = gv[pl.ds(q * 16, 16)]
        for t in range(16):
          i = q * 16 + t
          b = bvec[t]
          d16 = pl.ds(0, 16)
          ge = jnp.exp(jnp.full((16,), gvec[t], _f32) - gmv[b, d16])
          dacc[b, d16] = dacc[b, d16] + ge
          for j in range(4):
            sl = pl.ds(j * 16, 16)
            aacc[b, sl] = aacc[b, sl] + ge * hv[i, sl]

    woff = pl.multiple_of(wid * SEG, 8)
    pltpu.sync_copy(aacc, oattn.at[pl.ds(woff, SEG)])
    pltpu.sync_copy(dacc, oden.at[pl.ds(woff, SEG)])

  f = pl.kernel(
      body,
      out_type=(
          jax.ShapeDtypeStruct((NC * NS * SEG, H), _f32),
          jax.ShapeDtypeStruct((NC * NS * SEG, 16), _f32),
      ),
      mesh=_sc_mesh(),
      compiler_params=pltpu.CompilerParams(use_tc_tiling_on_sc=False),
      scratch_types=[
          pltpu.VMEM((PCH, H), _f32),
          pltpu.VMEM((PCH,), _f32),
          pltpu.VMEM((PCH,), jnp.int32),
          pltpu.VMEM((SEG, 16), _f32),
          pltpu.VMEM((SEG, H), _f32),
          pltpu.VMEM((SEG, 16), _f32),
      ],
  )
  return f(h, gate, batch2, gmax)


def _full(spec_shape, rank_map=None):
  return pl.BlockSpec(spec_shape, rank_map or (lambda i: (0,) * len(spec_shape)))


def _dot(a, b):
  return jnp.dot(a, b, preferred_element_type=_f32)


def _edge_proj(eap, W1, b1, W2, b2, lwstack, lbstack):
  Eb = 512
  grid = (E_PAD // Eb,)

  def body(ear, W1r, b1r, W2r, b2r, lwr, lbr, o0, o1, o2, o3, o4):
    ea = ear[...]
    enc = jnp.maximum(_dot(ea, W1r[...]) + b1r[...], 0.0)
    enc = _dot(enc, W2r[...]) + b2r[...]
    for l, o in enumerate((o0, o1, o2, o3, o4)):
      rl = _dot(enc, lwr[l]) + lbr[l]
      o[0] = rl[:, :32]
      o[1] = rl[:, 32:]

  return pl.pallas_call(
      body,
      grid=grid,
      in_specs=[
          pl.BlockSpec((Eb, FE), lambda i: (i, 0)),
          _full((FE, H)), _full((1, H)), _full((H, H)), _full((1, H)),
          _full((5, H, H)), _full((5, 1, H)),
      ],
      out_specs=tuple(
          pl.BlockSpec((2, Eb, 32), lambda i: (0, i, 0)) for _ in range(5)
      ),
      out_shape=tuple(
          jax.ShapeDtypeStruct((2, E_PAD, 32), _f32) for _ in range(5)
      ),
  )(eap, W1, b1, W2, b2, lwstack, lbstack)


def _node(h, agg, W1, b1, W2, b2, gam, bet, res):
  Nb = 1024
  grid = (N_PAD // Nb,)

  def body(hr, ar, W1r, b1r, W2r, b2r, gr, br, rr, ho):
    hcat = jnp.concatenate([hr[0], hr[1]], axis=1)
    xa = hcat + jnp.concatenate([ar[0], ar[1]], axis=1)
    t = jnp.maximum(_dot(xa, W1r[...]) + b1r[...], 0.0)
    t = _dot(t, W2r[...]) + b2r[...]
    t = jnp.maximum(t * gr[...] + br[...], 0.0)
    hn = rr[...] * hcat + t
    ho[0] = hn[:, :32]
    ho[1] = hn[:, 32:]

  return pl.pallas_call(
      body,
      grid=grid,
      in_specs=[
          pl.BlockSpec((2, Nb, 32), lambda i: (0, i, 0)),
          pl.BlockSpec((2, Nb, 32), lambda i: (0, i, 0)),
          _full((H, H)), _full((1, H)), _full((H, H)), _full((1, H)),
          _full((1, H)), _full((1, H)), _full((1, 1)),
      ],
      out_specs=pl.BlockSpec((2, Nb, 32), lambda i: (0, i, 0)),
      out_shape=jax.ShapeDtypeStruct((2, N_PAD, 32), _f32),
  )(h, agg, W1, b1, W2, b2, gam, bet, res)


def _gate_cat(h, gW1, gb1, gW2, gb2):
  Nb = 1024
  grid = (N_PAD // Nb,)

  def body(hr, gW1r, gb1r, gW2r, gb2r, hco, gto):
    hcat = jnp.concatenate([hr[0], hr[1]], axis=1)
    hco[...] = hcat
    gt = jnp.maximum(_dot(hcat, gW1r[...]) + gb1r[...], 0.0)
    gto[...] = _dot(gt, gW2r[...]) + gb2r[...]

  return pl.pallas_call(
      body,
      grid=grid,
      in_specs=[
          pl.BlockSpec((2, Nb, 32), lambda i: (0, i, 0)),
          _full((H, H)), _full((1, H)), _full((H, 1)), _full((1, 1)),
      ],
      out_specs=(
          pl.BlockSpec((Nb, H), lambda i: (i, 0)),
          pl.BlockSpec((Nb, 1), lambda i: (i, 0)),
      ),
      out_shape=(
          jax.ShapeDtypeStruct((N_PAD, H), _f32),
          jax.ShapeDtypeStruct((N_PAD, 1), _f32),
      ),
  )(h, gW1, gb1, gW2, gb2)


def _merge1(psum, pmax, pgmax, pcnt):
  grid = (NC * NS,)

  def body(sr, mr, gr, cr, osum, omax, ogmax, ocnt):
    i = pl.program_id(0)

    @pl.when(i == 0)
    def _():
      osum[...] = jnp.zeros_like(osum)
      omax[...] = jnp.full_like(omax, NEG)
      ogmax[...] = jnp.full_like(ogmax, NEG)
      ocnt[...] = jnp.zeros_like(ocnt)

    osum[...] += sr[...]
    omax[...] = jnp.maximum(omax[...], mr[...])
    ogmax[...] = jnp.maximum(ogmax[...], gr[...])
    ocnt[...] += cr[...]

  return pl.pallas_call(
      body,
      grid=grid,
      in_specs=[
          pl.BlockSpec((SEG, H), lambda i: (i, 0)),
          pl.BlockSpec((SEG, H), lambda i: (i, 0)),
          pl.BlockSpec((SEG, 16), lambda i: (i, 0)),
          pl.BlockSpec((SEG, 16), lambda i: (i, 0)),
      ],
      out_specs=(
          pl.BlockSpec((SEG, H), lambda i: (0, 0)),
          pl.BlockSpec((SEG, H), lambda i: (0, 0)),
          pl.BlockSpec((SEG, 16), lambda i: (0, 0)),
          pl.BlockSpec((SEG, 16), lambda i: (0, 0)),
      ),
      out_shape=(
          jax.ShapeDtypeStruct((SEG, H), _f32),
          jax.ShapeDtypeStruct((SEG, H), _f32),
          jax.ShapeDtypeStruct((SEG, 16), _f32),
          jax.ShapeDtypeStruct((SEG, 16), _f32),
      ),
      compiler_params=pltpu.CompilerParams(
          dimension_semantics=("arbitrary",)
      ),
  )(psum, pmax, pgmax, pcnt)


def _merge2(pattn, pden):
  grid = (NC * NS,)

  def body(ar, dr, oattn, oden):
    i = pl.program_id(0)

    @pl.when(i == 0)
    def _():
      oattn[...] = jnp.zeros_like(oattn)
      oden[...] = jnp.zeros_like(oden)

    oattn[...] += ar[...]
    oden[...] += dr[...]

  return pl.pallas_call(
      body,
      grid=grid,
      in_specs=[
          pl.BlockSpec((SEG, H), lambda i: (i, 0)),
          pl.BlockSpec((SEG, 16), lambda i: (i, 0)),
      ],
      out_specs=(
          pl.BlockSpec((SEG, H), lambda i: (0, 0)),
          pl.BlockSpec((SEG, 16), lambda i: (0, 0)),
      ),
      out_shape=(
          jax.ShapeDtypeStruct((SEG, H), _f32),
          jax.ShapeDtypeStruct((SEG, 16), _f32),
      ),
      compiler_params=pltpu.CompilerParams(
          dimension_semantics=("arbitrary",)
      ),
  )(pattn, pden)


def _final(msum, mmax, mcnt, mattn, mden, cW, cb):
  def body(sr, mr, cr, ar, dr, cWr, cbr, out):
    s = sr[pl.ds(0, B), :]
    mx = mr[pl.ds(0, B), :]
    cnt = cr[pl.ds(0, B), pl.ds(0, 1)]
    at = ar[pl.ds(0, B), :]
    dn = dr[pl.ds(0, B), pl.ds(0, 1)]
    mean = s / jnp.maximum(cnt, 1.0)
    mxo = jnp.where(cnt > 0.0, mx, 0.0)
    ato = at / (dn + 1e-16)
    feat = jnp.concatenate([mean, s, mxo, ato], axis=1)
    out[...] = _dot(feat, cWr[...]) + cbr[...]

  return pl.pallas_call(
      body,
      grid=(1,),
      in_specs=[
          _full((SEG, H)), _full((SEG, H)), _full((SEG, 16)),
          _full((SEG, H)), _full((SEG, 16)),
          _full((4 * H, TASKS)), _full((1, TASKS)),
      ],
      out_specs=pl.BlockSpec((B, TASKS), lambda i: (0, 0)),
      out_shape=jax.ShapeDtypeStruct((B, TASKS), _f32),
  )(msum, mmax, mcnt, mattn, mden, cW, cb)


def kernel(x, edge_attr, edge_index, batch, params):
  p = params
  src1 = jnp.pad(edge_index[0], (0, E_PAD - E))
  dst1 = jnp.pad(edge_index[1], (0, E_PAD - E), constant_values=N)
  eap = jnp.pad(edge_attr, ((0, E_PAD - E), (0, 0)))
  batch2 = jnp.pad(batch, (0, N_PAD - N), constant_values=B)

  h0 = jnp.stack([
      jnp.pad(x, ((0, N_PAD - N), (0, 32 - FN))),
      jnp.zeros((N_PAD, 32), _f32),
  ])
  lw0p = jnp.pad(p["conv0_lin_W"], ((0, 0), (0, H - FN)))
  lb0p = jnp.pad(p["conv0_lin_b"], (0, H - FN))
  W1_0p = jnp.pad(p["conv0_W1"], ((0, H - FN), (0, 0)))

  lwstack = jnp.stack([lw0p] + [p["conv%d_lin_W" % l] for l in range(1, 5)])
  lbstack = jnp.stack(
      [lb0p.reshape(1, H)]
      + [p["conv%d_lin_b" % l].reshape(1, H) for l in range(1, 5)]
  )
  W1s = [W1_0p] + [p["conv%d_W1" % l] for l in range(1, 5)]
  b1s = [p["conv%d_b1" % l].reshape(1, H) for l in range(5)]
  W2s = [p["conv%d_W2" % l] for l in range(5)]
  b2s = [p["conv%d_b2" % l].reshape(1, H) for l in range(5)]
  gams = [(p["bn%d_gamma" % l] / jnp.sqrt(1.0 + BN_EPS)).reshape(1, H)
          for l in range(5)]
  bets = [p["bn%d_beta" % l].reshape(1, H) for l in range(5)]
  res = [jnp.full((1, 1), float(l > 0), _f32) for l in range(5)]

  es = _edge_proj(
      eap, p["edge_mlp_W1"], p["edge_mlp_b1"].reshape(1, H),
      p["edge_mlp_W2"], p["edge_mlp_b2"].reshape(1, H), lwstack, lbstack
  )

  h = h0
  for l in range(5):
    agg = _edge_sc(h, es[l], src1, dst1)
    h = _node(h, agg, W1s[l], b1s[l], W2s[l], b2s[l], gams[l], bets[l],
              res[l])

  hcat, gate = _gate_cat(
      h, p["gate_W1"], p["gate_b1"].reshape(1, H),
      p["gate_W2"], p["gate_b2"].reshape(1, 1),
  )
  gate1 = gate.reshape(N_PAD)

  psum, pmax, pgmax, pcnt = _pool1(hcat, gate1, batch2)
  msum, mmax, mgmax, mcnt = _merge1(psum, pmax, pgmax, pcnt)
  pattn, pden = _pool2(hcat, gate1, batch2, mgmax)
  mattn, mden = _merge2(pattn, pden)
  return _final(msum, mmax, mcnt, mattn, mden,
                p["cls_W"], p["cls_b"].reshape(1, TASKS))


# speedup vs baseline: 1.5392x; 1.2302x over previous
import functools

import jax
import jax.numpy as jnp
from jax import lax
from jax.experimental import pallas as pl
from jax.experimental.pallas import tpu as pltpu
from jax.experimental.pallas import tpu_sc as plsc

N = 50000
E = 800000
FN = 9
FE = 3
H = 64
B = 256
TASKS = 12
BN_EPS = 1e-5

NC, NS = 2, 16
N_PAD = 50176
E_PAD = 819200
CHUNK = 256
GRP = CHUNK // 128
ROWS_T = N_PAD // (NC * NS)
PCH = 224
SEG = 264
NEG = -3.4028235e38

_f32 = jnp.float32


def _sc_mesh():
  return plsc.VectorSubcoreMesh(
      core_axis_name="c", subcore_axis_name="s", num_cores=NC, num_subcores=NS
  )


D = 32
PER_SUB = E_PAD // NS
N_CHUNKS = PER_SUB // CHUNK
ROWS_SUB = N_PAD // NS


def _edge_body(tab, eproj, src1, dst1, out, sidx0, sidx1, didx0, didx1,
               xg, ev, acc, lsem, esem, gsem, ssem):
  c = lax.axis_index("c")
  s = lax.axis_index("s")

  zero = jnp.zeros((16,), _f32)

  @pl.loop(0, CHUNK)
  def _(i):
    for j in range(D // 16):
      xg[i, pl.ds(j * 16, 16)] = zero

  row0 = pl.multiple_of(s * ROWS_SUB, 64)
  nf = ROWS_SUB // CHUNK
  rem = ROWS_SUB - nf * CHUNK
  for k in range(nf):
    pltpu.sync_copy(xg, acc.at[pl.ds(row0 + k * CHUNK, CHUNK)])
  if rem:
    pltpu.sync_copy(
        xg.at[pl.ds(0, rem)], acc.at[pl.ds(row0 + nf * CHUNK, rem)]
    )
  plsc.subcore_barrier()

  tabc = tab.at[c]
  ec_ref = eproj.at[c]
  ebase = pl.multiple_of(s * PER_SUB, 1024)

  @pl.loop(0, N_CHUNKS)
  def _(k):
    eoff = pl.multiple_of(ebase + k * CHUNK, CHUNK)
    idx_loads = [
        pltpu.make_async_copy(src1.at[pl.ds(eoff, 128)], sidx0, lsem),
        pltpu.make_async_copy(src1.at[pl.ds(eoff + 128, 128)], sidx1, lsem),
        pltpu.make_async_copy(dst1.at[pl.ds(eoff, 128)], didx0, lsem),
        pltpu.make_async_copy(dst1.at[pl.ds(eoff + 128, 128)], didx1, lsem),
    ]
    e_load = pltpu.make_async_copy(ec_ref.at[pl.ds(eoff, CHUNK)], ev, esem)
    for d in idx_loads:
      d.start()
    e_load.start()
    for d in idx_loads:
      d.wait()
    gathers = [
        pltpu.make_async_copy(tabc.at[sidx0], xg.at[pl.ds(0, 128)], gsem),
        pltpu.make_async_copy(tabc.at[sidx1], xg.at[pl.ds(128, 128)], gsem),
    ]
    for d in gathers:
      d.start()
    e_load.wait()
    for d in gathers:
      d.wait()

    @pl.loop(0, CHUNK, unroll=8)
    def _(i):
      for j in range(D // 16):
        sl = pl.ds(j * 16, 16)
        ev[i, sl] = jnp.maximum(xg[i, sl] + ev[i, sl], 0.0)

    scatters = [
        pltpu.make_async_copy(ev.at[pl.ds(0, 128)], acc.at[didx0], ssem),
        pltpu.make_async_copy(ev.at[pl.ds(128, 128)], acc.at[didx1], ssem),
    ]
    for d in scatters:
      d.start(add=True)
    for d in scatters:
      d.wait()

  plsc.subcore_barrier()
  pltpu.sync_copy(
      acc.at[pl.ds(row0, ROWS_SUB)],
      out.at[c].at[pl.ds(row0, ROWS_SUB)],
  )


_edge_sc = pl.kernel(
    _edge_body,
    out_type=jax.ShapeDtypeStruct((NC, N_PAD, D), _f32),
    mesh=_sc_mesh(),
    compiler_params=pltpu.CompilerParams(use_tc_tiling_on_sc=False),
    scratch_types=[
        pltpu.VMEM((128,), jnp.int32),
        pltpu.VMEM((128,), jnp.int32),
        pltpu.VMEM((128,), jnp.int32),
        pltpu.VMEM((128,), jnp.int32),
        pltpu.VMEM((CHUNK, D), _f32),
        pltpu.VMEM((CHUNK, D), _f32),
        pltpu.VMEM_SHARED((N_PAD, D), _f32),
        pltpu.SemaphoreType.DMA,
        pltpu.SemaphoreType.DMA,
        pltpu.SemaphoreType.DMA,
        pltpu.SemaphoreType.DMA,
    ],
)


def _pool1(h, gate, batch2):
  def body(hh, gg, bb, osum, omax, ogmax, ocnt, hv, gv, bv, sacc, macc, gacc,
           cacc):
    c = lax.axis_index("c")
    s = lax.axis_index("s")
    wid = c * NS + s
    base = wid * ROWS_T

    zero = jnp.zeros((16,), _f32)
    neg = jnp.full((16,), NEG, _f32)

    @pl.loop(0, SEG)
    def _(i):
      for j in range(4):
        sacc[i, pl.ds(j * 16, 16)] = zero
        macc[i, pl.ds(j * 16, 16)] = neg
      cacc[i, pl.ds(0, 16)] = zero
      gacc[i, pl.ds(0, 16)] = neg

    @pl.loop(0, ROWS_T // PCH)
    def _(k):
      off = pl.multiple_of(base + k * PCH, 32)
      pltpu.sync_copy(hh.at[pl.ds(off, PCH)], hv)
      pltpu.sync_copy(gg.at[pl.ds(off, PCH)], gv)
      pltpu.sync_copy(bb.at[pl.ds(off, PCH)], bv)

      @pl.loop(0, PCH // 16)
      def _(q):
        bvec = bv[pl.ds(q * 16, 16)]
        gvec = gv[pl.ds(q * 16, 16)]
        for t in range(16):
          i = q * 16 + t
          b = bvec[t]
          for j in range(4):
            sl = pl.ds(j * 16, 16)
            hvec = hv[i, sl]
            sacc[b, sl] = sacc[b, sl] + hvec
            macc[b, sl] = jnp.maximum(macc[b, sl], hvec)
          c16 = pl.ds(0, 16)
          gvb = jnp.full((16,), gvec[t], _f32)
          gacc[b, c16] = jnp.maximum(gacc[b, c16], gvb)
          cacc[b, c16] = cacc[b, c16] + 1.0

    woff = pl.multiple_of(wid * SEG, 8)
    pltpu.sync_copy(sacc, osum.at[pl.ds(woff, SEG)])
    pltpu.sync_copy(macc, omax.at[pl.ds(woff, SEG)])
    pltpu.sync_copy(gacc, ogmax.at[pl.ds(woff, SEG)])
    pltpu.sync_copy(cacc, ocnt.at[pl.ds(woff, SEG)])

  f = pl.kernel(
      body,
      out_type=(
          jax.ShapeDtypeStruct((NC * NS * SEG, H), _f32),
          jax.ShapeDtypeStruct((NC * NS * SEG, H), _f32),
          jax.ShapeDtypeStruct((NC * NS * SEG, 16), _f32),
          jax.ShapeDtypeStruct((NC * NS * SEG, 16), _f32),
      ),
      mesh=_sc_mesh(),
      compiler_params=pltpu.CompilerParams(use_tc_tiling_on_sc=False),
      scratch_types=[
          pltpu.VMEM((PCH, H), _f32),
          pltpu.VMEM((PCH,), _f32),
          pltpu.VMEM((PCH,), jnp.int32),
          pltpu.VMEM((SEG, H), _f32),
          pltpu.VMEM((SEG, H), _f32),
          pltpu.VMEM((SEG, 16), _f32),
          pltpu.VMEM((SEG, 16), _f32),
      ],
  )
  return f(h, gate, batch2)


def _pool2(h, gate, batch2, gmax):
  def body(hh, gg, bb, gm, oattn, oden, hv, gv, bv, gmv, aacc, dacc):
    c = lax.axis_index("c")
    s = lax.axis_index("s")
    wid = c * NS + s
    base = wid * ROWS_T

    zero = jnp.zeros((16,), _f32)

    @pl.loop(0, SEG)
    def _(i):
      for j in range(4):
        aacc[i, pl.ds(j * 16, 16)] = zero
      dacc[i, pl.ds(0, 16)] = zero

    pltpu.sync_copy(gm, gmv)

    @pl.loop(0, ROWS_T // PCH)
    def _(k):
      off = pl.multiple_of(base + k * PCH, 32)
      pltpu.sync_copy(hh.at[pl.ds(off, PCH)], hv)
      pltpu.sync_copy(gg.at[pl.ds(off, PCH)], gv)
      pltpu.sync_copy(bb.at[pl.ds(off, PCH)], bv)

      @pl.loop(0, PCH // 16)
      def _(q):
        bvec = bv[pl.ds(q * 16, 16)]
        gvec = gv[pl.ds(q * 16, 16)]
        for t in range(16):
          i = q * 16 + t
          b = bvec[t]
          d16 = pl.ds(0, 16)
          ge = jnp.exp(jnp.full((16,), gvec[t], _f32) - gmv[b, d16])
          dacc[b, d16] = dacc[b, d16] + ge
          for j in range(4):
            sl = pl.ds(j * 16, 16)
            aacc[b, sl] = aacc[b, sl] + ge * hv[i, sl]

    woff = pl.multiple_of(wid * SEG, 8)
    pltpu.sync_copy(aacc, oattn.at[pl.ds(woff, SEG)])
    pltpu.sync_copy(dacc, oden.at[pl.ds(woff, SEG)])

  f = pl.kernel(
      body,
      out_type=(
          jax.ShapeDtypeStruct((NC * NS * SEG, H), _f32),
          jax.ShapeDtypeStruct((NC * NS * SEG, 16), _f32),
      ),
      mesh=_sc_mesh(),
      compiler_params=pltpu.CompilerParams(use_tc_tiling_on_sc=False),
      scratch_types=[
          pltpu.VMEM((PCH, H), _f32),
          pltpu.VMEM((PCH,), _f32),
          pltpu.VMEM((PCH,), jnp.int32),
          pltpu.VMEM((SEG, 16), _f32),
          pltpu.VMEM((SEG, H), _f32),
          pltpu.VMEM((SEG, 16), _f32),
      ],
  )
  return f(h, gate, batch2, gmax)


def _full(spec_shape, rank_map=None):
  return pl.BlockSpec(spec_shape, rank_map or (lambda i: (0,) * len(spec_shape)))


def _dot(a, b):
  return jnp.dot(a, b, preferred_element_type=_f32)


def _edge_proj(eap, W1, b1, W2, b2, lwstack, lbstack):
  Eb = 512
  grid = (E_PAD // Eb,)

  def body(ear, W1r, b1r, W2r, b2r, lwr, lbr, o0, o1, o2, o3, o4):
    ea = ear[...]
    enc = jnp.maximum(_dot(ea, W1r[...]) + b1r[...], 0.0)
    enc = _dot(enc, W2r[...]) + b2r[...]
    for l, o in enumerate((o0, o1, o2, o3, o4)):
      rl = _dot(enc, lwr[l]) + lbr[l]
      o[0] = rl[:, :32]
      o[1] = rl[:, 32:]

  return pl.pallas_call(
      body,
      grid=grid,
      in_specs=[
          pl.BlockSpec((Eb, FE), lambda i: (i, 0)),
          _full((FE, H)), _full((1, H)), _full((H, H)), _full((1, H)),
          _full((5, H, H)), _full((5, 1, H)),
      ],
      out_specs=tuple(
          pl.BlockSpec((2, Eb, 32), lambda i: (0, i, 0)) for _ in range(5)
      ),
      out_shape=tuple(
          jax.ShapeDtypeStruct((2, E_PAD, 32), _f32) for _ in range(5)
      ),
  )(eap, W1, b1, W2, b2, lwstack, lbstack)


def _node(h, agg, W1, b1, W2, b2, gam, bet, res):
  Nb = 1024
  grid = (N_PAD // Nb,)

  def body(hr, ar, W1r, b1r, W2r, b2r, gr, br, rr, ho):
    hcat = jnp.concatenate([hr[0], hr[1]], axis=1)
    xa = hcat + jnp.concatenate([ar[0], ar[1]], axis=1)
    t = jnp.maximum(_dot(xa, W1r[...]) + b1r[...], 0.0)
    t = _dot(t, W2r[...]) + b2r[...]
    t = jnp.maximum(t * gr[...] + br[...], 0.0)
    hn = rr[...] * hcat + t
    ho[0] = hn[:, :32]
    ho[1] = hn[:, 32:]

  return pl.pallas_call(
      body,
      grid=grid,
      in_specs=[
          pl.BlockSpec((2, Nb, 32), lambda i: (0, i, 0)),
          pl.BlockSpec((2, Nb, 32), lambda i: (0, i, 0)),
          _full((H, H)), _full((1, H)), _full((H, H)), _full((1, H)),
          _full((1, H)), _full((1, H)), _full((1, 1)),
      ],
      out_specs=pl.BlockSpec((2, Nb, 32), lambda i: (0, i, 0)),
      out_shape=jax.ShapeDtypeStruct((2, N_PAD, 32), _f32),
  )(h, agg, W1, b1, W2, b2, gam, bet, res)


def _gate_cat(h, gW1, gb1, gW2, gb2):
  Nb = 1024
  grid = (N_PAD // Nb,)

  def body(hr, gW1r, gb1r, gW2r, gb2r, hco, gto):
    hcat = jnp.concatenate([hr[0], hr[1]], axis=1)
    hco[...] = hcat
    gt = jnp.maximum(_dot(hcat, gW1r[...]) + gb1r[...], 0.0)
    gto[...] = _dot(gt, gW2r[...]) + gb2r[...]

  return pl.pallas_call(
      body,
      grid=grid,
      in_specs=[
          pl.BlockSpec((2, Nb, 32), lambda i: (0, i, 0)),
          _full((H, H)), _full((1, H)), _full((H, 1)), _full((1, 1)),
      ],
      out_specs=(
          pl.BlockSpec((Nb, H), lambda i: (i, 0)),
          pl.BlockSpec((Nb, 1), lambda i: (i, 0)),
      ),
      out_shape=(
          jax.ShapeDtypeStruct((N_PAD, H), _f32),
          jax.ShapeDtypeStruct((N_PAD, 1), _f32),
      ),
  )(h, gW1, gb1, gW2, gb2)


def _merge1(psum, pmax, pgmax, pcnt):
  grid = (NC * NS,)

  def body(sr, mr, gr, cr, osum, omax, ogmax, ocnt):
    i = pl.program_id(0)

    @pl.when(i == 0)
    def _():
      osum[...] = jnp.zeros_like(osum)
      omax[...] = jnp.full_like(omax, NEG)
      ogmax[...] = jnp.full_like(ogmax, NEG)
      ocnt[...] = jnp.zeros_like(ocnt)

    osum[...] += sr[...]
    omax[...] = jnp.maximum(omax[...], mr[...])
    ogmax[...] = jnp.maximum(ogmax[...], gr[...])
    ocnt[...] += cr[...]

  return pl.pallas_call(
      body,
      grid=grid,
      in_specs=[
          pl.BlockSpec((SEG, H), lambda i: (i, 0)),
          pl.BlockSpec((SEG, H), lambda i: (i, 0)),
          pl.BlockSpec((SEG, 16), lambda i: (i, 0)),
          pl.BlockSpec((SEG, 16), lambda i: (i, 0)),
      ],
      out_specs=(
          pl.BlockSpec((SEG, H), lambda i: (0, 0)),
          pl.BlockSpec((SEG, H), lambda i: (0, 0)),
          pl.BlockSpec((SEG, 16), lambda i: (0, 0)),
          pl.BlockSpec((SEG, 16), lambda i: (0, 0)),
      ),
      out_shape=(
          jax.ShapeDtypeStruct((SEG, H), _f32),
          jax.ShapeDtypeStruct((SEG, H), _f32),
          jax.ShapeDtypeStruct((SEG, 16), _f32),
          jax.ShapeDtypeStruct((SEG, 16), _f32),
      ),
      compiler_params=pltpu.CompilerParams(
          dimension_semantics=("arbitrary",)
      ),
  )(psum, pmax, pgmax, pcnt)


def _merge2(pattn, pden):
  grid = (NC * NS,)

  def body(ar, dr, oattn, oden):
    i = pl.program_id(0)

    @pl.when(i == 0)
    def _():
      oattn[...] = jnp.zeros_like(oattn)
      oden[...] = jnp.zeros_like(oden)

    oattn[...] += ar[...]
    oden[...] += dr[...]

  return pl.pallas_call(
      body,
      grid=grid,
      in_specs=[
          pl.BlockSpec((SEG, H), lambda i: (i, 0)),
          pl.BlockSpec((SEG, 16), lambda i: (i, 0)),
      ],
      out_specs=(
          pl.BlockSpec((SEG, H), lambda i: (0, 0)),
          pl.BlockSpec((SEG, 16), lambda i: (0, 0)),
      ),
      out_shape=(
          jax.ShapeDtypeStruct((SEG, H), _f32),
          jax.ShapeDtypeStruct((SEG, 16), _f32),
      ),
      compiler_params=pltpu.CompilerParams(
          dimension_semantics=("arbitrary",)
      ),
  )(pattn, pden)


def _final(msum, mmax, mcnt, mattn, mden, cW, cb):
  def body(sr, mr, cr, ar, dr, cWr, cbr, out):
    s = sr[pl.ds(0, B), :]
    mx = mr[pl.ds(0, B), :]
    cnt = cr[pl.ds(0, B), pl.ds(0, 1)]
    at = ar[pl.ds(0, B), :]
    dn = dr[pl.ds(0, B), pl.ds(0, 1)]
    mean = s / jnp.maximum(cnt, 1.0)
    mxo = jnp.where(cnt > 0.0, mx, 0.0)
    ato = at / (dn + 1e-16)
    feat = jnp.concatenate([mean, s, mxo, ato], axis=1)
    out[...] = _dot(feat, cWr[...]) + cbr[...]

  return pl.pallas_call(
      body,
      grid=(1,),
      in_specs=[
          _full((SEG, H)), _full((SEG, H)), _full((SEG, 16)),
          _full((SEG, H)), _full((SEG, 16)),
          _full((4 * H, TASKS)), _full((1, TASKS)),
      ],
      out_specs=pl.BlockSpec((B, TASKS), lambda i: (0, 0)),
      out_shape=jax.ShapeDtypeStruct((B, TASKS), _f32),
  )(msum, mmax, mcnt, mattn, mden, cW, cb)


def kernel(x, edge_attr, edge_index, batch, params):
  p = params
  src1 = jnp.pad(edge_index[0], (0, E_PAD - E))
  dst1 = jnp.pad(edge_index[1], (0, E_PAD - E), constant_values=N)
  eap = jnp.pad(edge_attr, ((0, E_PAD - E), (0, 0)))
  batch2 = jnp.pad(batch, (0, N_PAD - N), constant_values=B)

  h0 = jnp.stack([
      jnp.pad(x, ((0, N_PAD - N), (0, 32 - FN))),
      jnp.zeros((N_PAD, 32), _f32),
  ])
  lw0p = jnp.pad(p["conv0_lin_W"], ((0, 0), (0, H - FN)))
  lb0p = jnp.pad(p["conv0_lin_b"], (0, H - FN))
  W1_0p = jnp.pad(p["conv0_W1"], ((0, H - FN), (0, 0)))

  lwstack = jnp.stack([lw0p] + [p["conv%d_lin_W" % l] for l in range(1, 5)])
  lbstack = jnp.stack(
      [lb0p.reshape(1, H)]
      + [p["conv%d_lin_b" % l].reshape(1, H) for l in range(1, 5)]
  )
  W1s = [W1_0p] + [p["conv%d_W1" % l] for l in range(1, 5)]
  b1s = [p["conv%d_b1" % l].reshape(1, H) for l in range(5)]
  W2s = [p["conv%d_W2" % l] for l in range(5)]
  b2s = [p["conv%d_b2" % l].reshape(1, H) for l in range(5)]
  gams = [(p["bn%d_gamma" % l] / jnp.sqrt(1.0 + BN_EPS)).reshape(1, H)
          for l in range(5)]
  bets = [p["bn%d_beta" % l].reshape(1, H) for l in range(5)]
  res = [jnp.full((1, 1), float(l > 0), _f32) for l in range(5)]

  es = _edge_proj(
      eap, p["edge_mlp_W1"], p["edge_mlp_b1"].reshape(1, H),
      p["edge_mlp_W2"], p["edge_mlp_b2"].reshape(1, H), lwstack, lbstack
  )

  h = h0
  for l in range(5):
    agg = _edge_sc(h, es[l], src1, dst1)
    h = _node(h, agg, W1s[l], b1s[l], W2s[l], b2s[l], gams[l], bets[l],
              res[l])

  hcat, gate = _gate_cat(
      h, p["gate_W1"], p["gate_b1"].reshape(1, H),
      p["gate_W2"], p["gate_b2"].reshape(1, 1),
  )
  gate1 = gate.reshape(N_PAD)

  psum, pmax, pgmax, pcnt = _pool1(hcat, gate1, batch2)
  msum, mmax, mgmax, mcnt = _merge1(psum, pmax, pgmax, pcnt)
  pattn, pden = _pool2(hcat, gate1, batch2, mgmax)
  mattn, mden = _merge2(pattn, pden)
  return _final(msum, mmax, mcnt, mattn, mden,
                p["cls_W"], p["cls_b"].reshape(1, TASKS))
